# Initial kernel scaffold; baseline (speedup 1.0000x reference)
import jax
import jax.numpy as jnp
from jax.experimental import pallas as pl


def kernel(feat, edge_index, W0, W1, W2, b2, g0, bt0, g1, bt1):
    raise NotImplementedError("write your pallas kernel here")


# speedup vs baseline: 3.5723x; 3.5723x over previous
import functools

import jax
import jax.numpy as jnp
from jax import lax
from jax.experimental import pallas as pl
from jax.experimental.pallas import tpu as pltpu
from jax.experimental.pallas import tpu_sc as plsc

N = 10000
E = 320000
D = 128
H = 128
C = 40

NC = 2
NS = 16
NW = NC * NS
EPW = E // NW
K = 80
NCHUNK = EPW // K
NP = 10240
RPT = NP // NS

_MESH = dict(core_axis_name="c", subcore_axis_name="s", num_cores=NC,
             num_subcores=NS)


def _deg_body(src_hbm, dst_hbm, z_hbm, ones_hbm, out_hbm,
              eidx, ones_v, acc):
    c = lax.axis_index("c")
    s = lax.axis_index("s")
    wid = s * NC + c
    ebase = wid * EPW
    r0 = s * RPT
    pltpu.sync_copy(ones_hbm, ones_v)
    for which, e_hbm in ((0, src_hbm), (1, dst_hbm)):
        pltpu.sync_copy(z_hbm.at[pl.ds(r0, RPT), :],
                        acc.at[pl.ds(r0, RPT), :])
        plsc.subcore_barrier()

        def body(i, carry):
            pltpu.sync_copy(e_hbm.at[pl.ds(ebase + i * K, K)], eidx)
            pltpu.sync_copy(ones_v, acc.at[eidx], add=True)
            return carry

        lax.fori_loop(0, NCHUNK, body, 0)
        plsc.subcore_barrier()
        pltpu.sync_copy(acc.at[pl.ds(r0, RPT), :],
                        out_hbm.at[which, c, pl.ds(r0, RPT), :])
        plsc.subcore_barrier()


def _degrees(src, dst, z128, ones):
    f = pl.kernel(
        _deg_body,
        out_type=jax.ShapeDtypeStruct((2, NC, NP, H), jnp.float32),
        mesh=plsc.VectorSubcoreMesh(**_MESH),
        scratch_types=[
            pltpu.VMEM((K,), jnp.int32),
            pltpu.VMEM((K, H), jnp.float32),
            pltpu.VMEM_SHARED((NP, H), jnp.float32),
        ],
    )
    return f(src, dst, z128, ones)


def _spmm_body(h_hbm, src_hbm, dst_hbm, z_hbm, out_hbm,
               sidx, didx, rows, acc, sem):
    c = lax.axis_index("c")
    s = lax.axis_index("s")
    wid = s * NC + c
    ebase = wid * EPW
    r0 = s * RPT
    pltpu.sync_copy(z_hbm.at[pl.ds(r0, RPT), :], acc.at[pl.ds(r0, RPT), :])
    plsc.subcore_barrier()

    def body(i, carry):
        base = ebase + i * K
        pltpu.sync_copy(src_hbm.at[pl.ds(base, K)], sidx)
        pltpu.sync_copy(dst_hbm.at[pl.ds(base, K)], didx)
        pltpu.async_copy(h_hbm.at[sidx], rows, sem).wait()
        pltpu.sync_copy(rows, acc.at[didx], add=True)
        return carry

    lax.fori_loop(0, NCHUNK, body, 0)
    plsc.subcore_barrier()
    pltpu.sync_copy(acc.at[pl.ds(r0, RPT), :],
                    out_hbm.at[c, pl.ds(r0, RPT), :])


def _spmm(h, src, dst, z128):
    f = pl.kernel(
        _spmm_body,
        out_type=jax.ShapeDtypeStruct((NC, NP, H), jnp.float32),
        mesh=plsc.VectorSubcoreMesh(**_MESH),
        scratch_types=[
            pltpu.VMEM((K,), jnp.int32),
            pltpu.VMEM((K,), jnp.int32),
            pltpu.VMEM((K, H), jnp.float32),
            pltpu.VMEM_SHARED((NP, H), jnp.float32),
            pltpu.SemaphoreType.DMA,
        ],
    )
    return f(h, src, dst, z128)


def _pre_body(feat_ref, degs_ref, h0_ref, osc_ref, isc_ref):
    scnt = degs_ref[0, 0, 0:N] + degs_ref[0, 1, 0:N]
    dcnt = degs_ref[1, 0, 0:N] + degs_ref[1, 1, 0:N]
    osc = lax.rsqrt(jnp.maximum(scnt[:, 0:1], 1.0))
    isc = lax.rsqrt(jnp.maximum(dcnt[:, 0:1], 1.0))
    h0_ref[...] = feat_ref[...] * osc
    osc_ref[...] = osc
    isc_ref[...] = isc


def _pre(feat, degs):
    return pl.pallas_call(
        _pre_body,
        out_shape=(
            jax.ShapeDtypeStruct((N, D), jnp.float32),
            jax.ShapeDtypeStruct((N, 1), jnp.float32),
            jax.ShapeDtypeStruct((N, 1), jnp.float32),
        ),
    )(feat, degs)


def _post_body(p_ref, isc_ref, osc_ref, wt_ref, g_ref, bt_ref, out_ref):
    r = (p_ref[0, 0:N] + p_ref[1, 0:N]) * isc_ref[...]
    y = jnp.dot(r, wt_ref[...], preferred_element_type=jnp.float32)
    mu = jnp.mean(y, axis=0, keepdims=True)
    yc = y - mu
    var = jnp.mean(yc * yc, axis=0, keepdims=True)
    h = jnp.maximum(yc * lax.rsqrt(var + 1e-5) * g_ref[...] + bt_ref[...],
                    0.0)
    out_ref[...] = h * osc_ref[...]


def _post(p, isc, osc, wt, g, bt):
    return pl.pallas_call(
        _post_body,
        out_shape=jax.ShapeDtypeStruct((N, H), jnp.float32),
    )(p, isc, osc, wt, g, bt)


def _final_body(p_ref, isc_ref, w2t_ref, b2_ref, out_ref):
    r = (p_ref[0, 0:N] + p_ref[1, 0:N]) * isc_ref[...]
    out_ref[...] = (jnp.dot(r, w2t_ref[...],
                            preferred_element_type=jnp.float32)
                    + b2_ref[...])


def _final(p, isc, w2t, b2):
    return pl.pallas_call(
        _final_body,
        out_shape=jax.ShapeDtypeStruct((N, C), jnp.float32),
    )(p, isc, w2t, b2)


def kernel(feat, edge_index, W0, W1, W2, b2, g0, bt0, g1, bt1):
    src = edge_index[0]
    dst = edge_index[1]
    z128 = jnp.zeros((NP, H), jnp.float32)
    ones = jnp.ones((K, H), jnp.float32)

    degs = _degrees(src, dst, z128, ones)
    h0, osc, isc = _pre(feat, degs)
    p = _spmm(h0, src, dst, z128)
    h1 = _post(p, isc, osc, W0.T, g0[None], bt0[None])
    p = _spmm(h1, src, dst, z128)
    h2 = _post(p, isc, osc, W1.T, g1[None], bt1[None])
    p = _spmm(h2, src, dst, z128)
    return _final(p, isc, W2.T, b2[None])


# speedup vs baseline: 7.9158x; 2.2159x over previous
import functools

import jax
import jax.numpy as jnp
from jax import lax
from jax.experimental import pallas as pl
from jax.experimental.pallas import tpu as pltpu
from jax.experimental.pallas import tpu_sc as plsc

N = 10000
E = 320000
D = 128
H = 128
C = 40

NC = 2
NS = 16
NW = NC * NS
EPW = E // NW
K = 40
NCHUNK = EPW // K
NBUF = 5
NP = 10240
RPT = NP // NS
GR = NP // H

_MESH = dict(core_axis_name="c", subcore_axis_name="s", num_cores=NC,
             num_subcores=NS)


def _deg_body(src_hbm, dst_hbm, z_hbm, ones_hbm, out_hbm,
              e0, e1, e2, e3, e4, ones_v, acc, *sems):
    c = lax.axis_index("c")
    s = lax.axis_index("s")
    wid = s * NC + c
    ebase = wid * EPW
    row0 = s * RPT
    eidx = (e0, e1, e2, e3, e4)
    dsem = sems[0:NBUF]
    ssem = sems[NBUF:2 * NBUF]
    pltpu.sync_copy(ones_hbm, ones_v)
    for which, e_hbm in ((0, src_hbm), (1, dst_hbm)):
        pltpu.sync_copy(z_hbm.at[pl.ds(row0, RPT), :],
                        acc.at[pl.ds(row0, RPT), :])
        plsc.subcore_barrier()

        def start(b, ch):
            pltpu.async_copy(e_hbm.at[pl.ds(ebase + ch * K, K)], eidx[b],
                             dsem[b])

        def wait_in(b):
            pltpu.make_async_copy(e_hbm.at[pl.ds(0, K)], eidx[b],
                                  dsem[b]).wait()

        for b in range(NBUF):
            start(b, b)

        @pl.loop(0, NCHUNK - NBUF, step=NBUF)
        def _(g):
            descs = []
            for b in range(NBUF):
                wait_in(b)
                descs.append(pltpu.async_copy(ones_v, acc.at[eidx[b]],
                                              ssem[b], add=True))
            for b in range(NBUF):
                descs[b].wait()
                start(b, g + NBUF + b)

        descs = []
        for b in range(NBUF):
            wait_in(b)
            descs.append(pltpu.async_copy(ones_v, acc.at[eidx[b]],
                                          ssem[b], add=True))
        for d in descs:
            d.wait()
        plsc.subcore_barrier()
        pltpu.sync_copy(acc.at[pl.ds(row0, RPT), :],
                        out_hbm.at[which, c, pl.ds(row0, RPT), :])
        plsc.subcore_barrier()


def _degrees(src, dst, z128, ones):
    f = pl.kernel(
        _deg_body,
        out_type=jax.ShapeDtypeStruct((2, NC, NP, H), jnp.float32),
        mesh=plsc.VectorSubcoreMesh(**_MESH),
        scratch_types=(
            [pltpu.VMEM((K,), jnp.int32) for _ in range(NBUF)]
            + [pltpu.VMEM((K, H), jnp.float32)]
            + [pltpu.VMEM_SHARED((NP, H), jnp.float32)]
            + [pltpu.SemaphoreType.DMA for _ in range(2 * NBUF)]
        ),
    )
    return f(src, dst, z128, ones)


def _spmm_body(h_hbm, src_hbm, dst_hbm, z_hbm, out_hbm,
               sidx, d0, d1, d2, d3, d4, r0b, r1b, r2b, r3b, r4b, acc,
               *sems):
    c = lax.axis_index("c")
    s = lax.axis_index("s")
    wid = s * NC + c
    ebase = wid * EPW
    row0 = s * RPT
    didx = (d0, d1, d2, d3, d4)
    rows = (r0b, r1b, r2b, r3b, r4b)
    dsem = sems[0:NBUF]
    gsem = sems[NBUF:2 * NBUF]
    ssem = sems[2 * NBUF:3 * NBUF]

    pltpu.sync_copy(z_hbm.at[pl.ds(row0, RPT), :],
                    acc.at[pl.ds(row0, RPT), :])
    pltpu.sync_copy(src_hbm.at[pl.ds(ebase, EPW)], sidx)
    plsc.subcore_barrier()

    def start(b, ch):
        pltpu.async_copy(dst_hbm.at[pl.ds(ebase + ch * K, K)], didx[b],
                         dsem[b])
        pltpu.async_copy(h_hbm.at[sidx.at[pl.ds(ch * K, K)]], rows[b],
                         gsem[b])

    def wait_in(b):
        pltpu.make_async_copy(dst_hbm.at[pl.ds(0, K)], didx[b],
                              dsem[b]).wait()
        pltpu.make_async_copy(h_hbm.at[sidx.at[pl.ds(0, K)]], rows[b],
                              gsem[b]).wait()

    for b in range(NBUF):
        start(b, b)

    @pl.loop(0, NCHUNK - NBUF, step=NBUF)
    def _(g):
        descs = []
        for b in range(NBUF):
            wait_in(b)
            descs.append(pltpu.async_copy(rows[b], acc.at[didx[b]],
                                          ssem[b], add=True))
        for b in range(NBUF):
            descs[b].wait()
            start(b, g + NBUF + b)

    descs = []
    for b in range(NBUF):
        wait_in(b)
        descs.append(pltpu.async_copy(rows[b], acc.at[didx[b]],
                                      ssem[b], add=True))
    for d in descs:
        d.wait()
    plsc.subcore_barrier()
    pltpu.sync_copy(acc.at[pl.ds(row0, RPT), :],
                    out_hbm.at[c, pl.ds(row0, RPT), :])


def _spmm(h, src, dst, z128):
    f = pl.kernel(
        _spmm_body,
        out_type=jax.ShapeDtypeStruct((NC, NP, H), jnp.float32),
        mesh=plsc.VectorSubcoreMesh(**_MESH),
        scratch_types=(
            [pltpu.VMEM((EPW,), jnp.int32)]
            + [pltpu.VMEM((K,), jnp.int32) for _ in range(NBUF)]
            + [pltpu.VMEM((K, H), jnp.float32) for _ in range(NBUF)]
            + [pltpu.VMEM_SHARED((NP, H), jnp.float32)]
            + [pltpu.SemaphoreType.DMA for _ in range(3 * NBUF)]
        ),
    )
    return f(h, src, dst, z128)


def _pre_body(feat_ref, degs_ref, h0_ref, osc_ref, isc_ref):
    scnt = degs_ref[0, 0, 0:N] + degs_ref[0, 1, 0:N]
    dcnt = degs_ref[1, 0, 0:N] + degs_ref[1, 1, 0:N]
    osc = lax.rsqrt(jnp.maximum(scnt, 1.0))
    isc = lax.rsqrt(jnp.maximum(dcnt, 1.0))
    h0_ref[...] = feat_ref[...] * osc
    osc_ref[...] = osc
    isc_ref[...] = isc


def _pre(feat, degs):
    return pl.pallas_call(
        _pre_body,
        out_shape=(
            jax.ShapeDtypeStruct((N, D), jnp.float32),
            jax.ShapeDtypeStruct((N, H), jnp.float32),
            jax.ShapeDtypeStruct((N, H), jnp.float32),
        ),
    )(feat, degs)


def _post_body(p_ref, isc_ref, osc_ref, wt_ref, g_ref, bt_ref, out_ref):
    r = (p_ref[0, 0:N] + p_ref[1, 0:N]) * isc_ref[...]
    y = jnp.dot(r, wt_ref[...], preferred_element_type=jnp.float32)
    mu = jnp.mean(y, axis=0, keepdims=True)
    yc = y - mu
    var = jnp.mean(yc * yc, axis=0, keepdims=True)
    h = jnp.maximum(yc * lax.rsqrt(var + 1e-5) * g_ref[...] + bt_ref[...],
                    0.0)
    out_ref[...] = h * osc_ref[...]


def _post(p, isc, osc, wt, g, bt):
    return pl.pallas_call(
        _post_body,
        out_shape=jax.ShapeDtypeStruct((N, H), jnp.float32),
    )(p, isc, osc, wt, g, bt)


def _final_body(p_ref, isc_ref, w2t_ref, b2_ref, out_ref):
    r = (p_ref[0, 0:N] + p_ref[1, 0:N]) * isc_ref[...]
    out_ref[...] = (jnp.dot(r, w2t_ref[...],
                            preferred_element_type=jnp.float32)
                    + b2_ref[...])


def _final(p, isc, w2t, b2):
    return pl.pallas_call(
        _final_body,
        out_shape=jax.ShapeDtypeStruct((N, C), jnp.float32),
    )(p, isc, w2t, b2)


def kernel(feat, edge_index, W0, W1, W2, b2, g0, bt0, g1, bt1):
    src = edge_index[0]
    dst = edge_index[1]
    z128 = jnp.zeros((NP, H), jnp.float32)
    ones = jnp.ones((K, H), jnp.float32)

    degs = _degrees(src, dst, z128, ones)
    h0, osc, isc = _pre(feat, degs)
    p = _spmm(h0, src, dst, z128)
    h1 = _post(p, isc, osc, W0.T, g0[None], bt0[None])
    p = _spmm(h1, src, dst, z128)
    h2 = _post(p, isc, osc, W1.T, g1[None], bt1[None])
    p = _spmm(h2, src, dst, z128)
    return _final(p, isc, W2.T, b2[None])
